# Initial kernel scaffold; baseline (speedup 1.0000x reference)
import jax
import jax.numpy as jnp
from jax.experimental import pallas as pl


def kernel(h, edge_index, W1, W2, mlp_bn_gamma, mlp_bn_beta, apply_bn_gamma, apply_bn_beta, out_bn_gamma, out_bn_beta):
    raise NotImplementedError("write your pallas kernel here")


# speedup vs baseline: 4.2484x; 4.2484x over previous
import functools

import jax
import jax.numpy as jnp
from jax import lax
from jax.experimental import pallas as pl
from jax.experimental.pallas import tpu as pltpu
from jax.experimental.pallas import tpu_sc as plsc

_EPS = 1e-5


def _build_segment_partials(N, D, E):
    NC, NS = 2, 16
    NW = NC * NS
    EPW = E // NW
    assert E % NW == 0
    B = 80
    NB = EPW // B
    assert EPW % B == 0
    ZR = 48
    RPS = (N // (NS * ZR)) * ZR
    TAIL = N - RPS * NS
    assert RPS % ZR == 0 and TAIL % 8 == 0 and 0 <= TAIL < ZR

    mesh = plsc.VectorSubcoreMesh(core_axis_name="c", subcore_axis_name="s")

    @functools.partial(
        pl.kernel,
        out_type=(
            jax.ShapeDtypeStruct((N, D), jnp.float32),
            jax.ShapeDtypeStruct((N, D), jnp.float32),
        ),
        mesh=mesh,
        scratch_types=[
            pltpu.VMEM((B,), jnp.int32),
            pltpu.VMEM((B,), jnp.int32),
            pltpu.VMEM((B, D), jnp.float32),
            pltpu.VMEM((ZR, D), jnp.float32),
            pltpu.VMEM_SHARED((N, D), jnp.float32),
            pltpu.SemaphoreType.DMA,
        ],
    )
    def seg_kernel(h_hbm, src_hbm, dst_hbm, out0, out1,
                   idx_s, idx_d, rows, zbuf, acc, sem):
        cid = lax.axis_index("c")
        sid = lax.axis_index("s")
        wid = cid * NS + sid

        def zero_body(k, _):
            r = k // (D // 16)
            c = (k % (D // 16)) * 16
            zbuf[r, pl.ds(c, 16)] = jnp.zeros((16,), jnp.float32)
            return _
        lax.fori_loop(0, ZR * (D // 16), zero_body, 0)

        def zcopy_body(k, _):
            base = pl.multiple_of(sid * RPS + k * ZR, 8)
            pltpu.sync_copy(zbuf, acc.at[pl.ds(base, ZR)])
            return _
        lax.fori_loop(0, RPS // ZR, zcopy_body, 0)
        if TAIL:
            @pl.when(sid == 0)
            def _():
                pltpu.sync_copy(zbuf.at[pl.ds(0, TAIL)],
                                acc.at[pl.ds(NS * RPS, TAIL)])
        plsc.subcore_barrier()

        def edge_body(j, _):
            base = pl.multiple_of(wid * EPW + j * B, 8)
            pltpu.sync_copy(src_hbm.at[pl.ds(base, B)], idx_s)
            pltpu.sync_copy(dst_hbm.at[pl.ds(base, B)], idx_d)
            pltpu.async_copy(h_hbm.at[idx_s], rows, sem).wait()
            pltpu.sync_copy(rows, acc.at[idx_d], add=True)
            return _
        lax.fori_loop(0, NB, edge_body, 0)
        plsc.subcore_barrier()

        rbase = pl.multiple_of(sid * RPS, 8)

        @pl.when(cid == 0)
        def _():
            pltpu.sync_copy(acc.at[pl.ds(rbase, RPS)], out0.at[pl.ds(rbase, RPS)])
            if TAIL:
                @pl.when(sid == 0)
                def _():
                    pltpu.sync_copy(acc.at[pl.ds(NS * RPS, TAIL)],
                                    out0.at[pl.ds(NS * RPS, TAIL)])

        @pl.when(cid == 1)
        def _():
            pltpu.sync_copy(acc.at[pl.ds(rbase, RPS)], out1.at[pl.ds(rbase, RPS)])
            if TAIL:
                @pl.when(sid == 0)
                def _():
                    pltpu.sync_copy(acc.at[pl.ds(NS * RPS, TAIL)],
                                    out1.at[pl.ds(NS * RPS, TAIL)])

    return seg_kernel


_R = 1000


def _dot(a, b):
    return jax.lax.dot_general(
        a, b, (((1,), (0,)), ((), ())),
        preferred_element_type=jnp.float32)


def _accum_stats(i, z, st_ref, acc_ref):
    blk = jnp.concatenate(
        [jnp.sum(z, axis=0)[None], jnp.sum(z * z, axis=0)[None]], axis=0)

    @pl.when(i == 0)
    def _():
        acc_ref[...] = blk

    @pl.when(i > 0)
    def _():
        acc_ref[...] = acc_ref[...] + blk

    @pl.when(i == pl.num_programs(0) - 1)
    def _():
        st_ref[...] = acc_ref[...]


def _bn_coeffs(st, gamma, beta, n):
    mean = st[0] / n
    var = st[1] / n - mean * mean
    inv = gamma[0] / jnp.sqrt(var + _EPS)
    shift = beta[0] - mean * inv
    return inv, shift


def _stage_a(h, p0, p1, wT):
    N, D = h.shape
    G = N // _R

    def body(x_ref, p0_ref, p1_ref, w_ref, z_ref, st_ref, acc_ref):
        i = pl.program_id(0)
        rst = x_ref[...] + p0_ref[...] + p1_ref[...]
        z = _dot(rst, w_ref[...])
        z_ref[...] = z
        _accum_stats(i, z, st_ref, acc_ref)

    row_spec = pl.BlockSpec((_R, D), lambda i: (i, 0))
    full_spec = pl.BlockSpec((D, D), lambda i: (0, 0))
    st_spec = pl.BlockSpec((2, D), lambda i: (0, 0))
    return pl.pallas_call(
        body,
        grid=(G,),
        in_specs=[row_spec, row_spec, row_spec, full_spec],
        out_specs=(row_spec, st_spec),
        out_shape=(jax.ShapeDtypeStruct((N, D), jnp.float32),
                   jax.ShapeDtypeStruct((2, D), jnp.float32)),
        scratch_shapes=[pltpu.VMEM((2, D), jnp.float32)],
    )(h, p0, p1, wT)


def _stage_b(z1, st1, gamma, beta, wT):
    N, D = z1.shape
    G = N // _R

    def body(x_ref, s_ref, g_ref, b_ref, w_ref, z_ref, st_ref, acc_ref):
        i = pl.program_id(0)
        inv, shift = _bn_coeffs(s_ref[...], g_ref[...], b_ref[...], N)
        u = jnp.maximum(x_ref[...] * inv[None] + shift[None], 0.0)
        z = _dot(u, w_ref[...])
        z_ref[...] = z
        _accum_stats(i, z, st_ref, acc_ref)

    row_spec = pl.BlockSpec((_R, D), lambda i: (i, 0))
    st_spec = pl.BlockSpec((2, D), lambda i: (0, 0))
    vec_spec = pl.BlockSpec((1, D), lambda i: (0, 0))
    full_spec = pl.BlockSpec((D, D), lambda i: (0, 0))
    return pl.pallas_call(
        body,
        grid=(G,),
        in_specs=[row_spec, st_spec, vec_spec, vec_spec, full_spec],
        out_specs=(row_spec, st_spec),
        out_shape=(jax.ShapeDtypeStruct((N, D), jnp.float32),
                   jax.ShapeDtypeStruct((2, D), jnp.float32)),
        scratch_shapes=[pltpu.VMEM((2, D), jnp.float32)],
    )(z1, st1, gamma, beta, wT)


def _stage_c(z2, st2, gamma, beta):
    N, D = z2.shape
    G = N // _R

    def body(x_ref, s_ref, g_ref, b_ref, v_ref, st_ref, acc_ref):
        i = pl.program_id(0)
        inv, shift = _bn_coeffs(s_ref[...], g_ref[...], b_ref[...], N)
        v = jnp.maximum(x_ref[...] * inv[None] + shift[None], 0.0)
        v_ref[...] = v
        _accum_stats(i, v, st_ref, acc_ref)

    row_spec = pl.BlockSpec((_R, D), lambda i: (i, 0))
    st_spec = pl.BlockSpec((2, D), lambda i: (0, 0))
    vec_spec = pl.BlockSpec((1, D), lambda i: (0, 0))
    return pl.pallas_call(
        body,
        grid=(G,),
        in_specs=[row_spec, st_spec, vec_spec, vec_spec],
        out_specs=(row_spec, st_spec),
        out_shape=(jax.ShapeDtypeStruct((N, D), jnp.float32),
                   jax.ShapeDtypeStruct((2, D), jnp.float32)),
        scratch_shapes=[pltpu.VMEM((2, D), jnp.float32)],
    )(z2, st2, gamma, beta)


def _stage_d(v, st3, gamma, beta, relu):
    N, D = v.shape
    G = N // _R

    def body(x_ref, s_ref, g_ref, b_ref, o_ref):
        inv, shift = _bn_coeffs(s_ref[...], g_ref[...], b_ref[...], N)
        z = x_ref[...] * inv[None] + shift[None]
        if relu:
            z = jnp.maximum(z, 0.0)
        o_ref[...] = z

    row_spec = pl.BlockSpec((_R, D), lambda i: (i, 0))
    st_spec = pl.BlockSpec((2, D), lambda i: (0, 0))
    vec_spec = pl.BlockSpec((1, D), lambda i: (0, 0))
    return pl.pallas_call(
        body,
        grid=(G,),
        in_specs=[row_spec, st_spec, vec_spec, vec_spec],
        out_specs=row_spec,
        out_shape=jax.ShapeDtypeStruct((N, D), jnp.float32),
    )(v, st3, gamma, beta)


def kernel(h, edge_index, W1, W2, mlp_bn_gamma, mlp_bn_beta,
           apply_bn_gamma, apply_bn_beta, out_bn_gamma, out_bn_beta):
    N, D = h.shape
    E = edge_index.shape[1]
    L = W1.shape[0]
    src = edge_index[0]
    dst = edge_index[1]
    seg = _build_segment_partials(N, D, E)

    for i in range(L):
        p0, p1 = seg(h, src, dst)
        z1, s1 = _stage_a(h, p0, p1, W1[i].T)
        z2, s2 = _stage_b(z1, s1, mlp_bn_gamma[i][None], mlp_bn_beta[i][None],
                          W2[i].T)
        v, s3 = _stage_c(z2, s2, apply_bn_gamma[i][None], apply_bn_beta[i][None])
        h = _stage_d(v, s3, out_bn_gamma[i][None], out_bn_beta[i][None],
                     relu=(i < L - 1))
    return h
